# baseline (speedup 1.0000x reference)
import functools

import jax
import jax.numpy as jnp
from jax import lax
from jax.experimental import pallas as pl
from jax.experimental import layout as jex_layout
from jax.experimental.pallas import tpu as pltpu
from jax.experimental.pallas import tpu_sc as plsc

_D = 64
_S = 200
_CHUNKS = (104, 96)
_NCHUNK = len(_CHUNKS)
_NC = 2
_NS = 16
_NW = _NC * _NS
_ROWLOOK = 4
_LANES = 16


@jax.jit
def _cbow_sc(ids, table):
    B = ids.shape[0]
    R = B // _NW

    mesh = plsc.VectorSubcoreMesh(core_axis_name="c", subcore_axis_name="s")

    @functools.partial(
        pl.kernel,
        out_type=jax.ShapeDtypeStruct((B, _D), jnp.float32),
        mesh=mesh,
        scratch_types=[
            pltpu.VMEM((R, _S), jnp.int32),
            pltpu.VMEM((R, _D), jnp.float32),
        ]
        + [
            pltpu.VMEM((_CHUNKS[c], _D), jnp.float32)
            for _ in range(_ROWLOOK)
            for c in range(_NCHUNK)
        ]
        + [pltpu.SemaphoreType.DMA for _ in range(_ROWLOOK * _NCHUNK)],
        compiler_params=pltpu.CompilerParams(
            use_tc_tiling_on_sc=False, needs_layout_passes=False
        ),
    )
    def cbow(ids_hbm, table_hbm, out_hbm, idx_v, out_v, *rest):
        nstg = _ROWLOOK * _NCHUNK
        bufs = rest[:nstg]
        sems = rest[nstg:]
        wid = lax.axis_index("s") * _NC + lax.axis_index("c")
        base = wid * R

        pltpu.sync_copy(ids_hbm.at[pl.ds(base, R)], idx_v)

        def issue(row, c, p):
            off = c * _CHUNKS[0]
            pltpu.async_copy(
                table_hbm.at[idx_v.at[row, pl.ds(off, _CHUNKS[c])]],
                bufs[p],
                sems[p],
            )

        def drain(c, p):
            pltpu.make_async_copy(
                table_hbm.at[idx_v.at[0, pl.ds(0, _CHUNKS[c])]],
                bufs[p],
                sems[p],
            ).wait()

        def reduce_buf(buf, n, accs):
            def body(jj, accs):
                a0, a1, a2, a3 = accs
                for u in range(4):
                    j = jj * 4 + u
                    a0 = a0 + buf[j, pl.ds(0, _LANES)]
                    a1 = a1 + buf[j, pl.ds(_LANES, _LANES)]
                    a2 = a2 + buf[j, pl.ds(2 * _LANES, _LANES)]
                    a3 = a3 + buf[j, pl.ds(3 * _LANES, _LANES)]
                return (a0, a1, a2, a3)

            return lax.fori_loop(0, n // 4, body, accs)

        scale = jnp.float32(1.0 / _S)

        for k in range(_ROWLOOK):
            for c in range(_NCHUNK):
                issue(k, c, k * _NCHUNK + c)

        def outer(i, _):
            r0 = i * _ROWLOOK
            for k in range(_ROWLOOK):
                r = r0 + k
                z = jnp.zeros((_LANES,), jnp.float32)
                accs = (z, z, z, z)
                for c in range(_NCHUNK):
                    p = k * _NCHUNK + c
                    drain(c, p)
                    accs = reduce_buf(bufs[p], _CHUNKS[c], accs)

                    @pl.when(r + _ROWLOOK < R)
                    def _():
                        issue(r + _ROWLOOK, c, p)

                a0, a1, a2, a3 = accs
                out_v[r, pl.ds(0, _LANES)] = a0 * scale
                out_v[r, pl.ds(_LANES, _LANES)] = a1 * scale
                out_v[r, pl.ds(2 * _LANES, _LANES)] = a2 * scale
                out_v[r, pl.ds(3 * _LANES, _LANES)] = a3 * scale
            return 0

        lax.fori_loop(0, R // _ROWLOOK, outer, 0)

        pltpu.sync_copy(out_v, out_hbm.at[pl.ds(base, R)])

    return cbow(ids, table)


def kernel(input_ids, table):
    table_rm = jex_layout.with_layout_constraint(
        table, jex_layout.Layout(major_to_minor=(0, 1))
    )
    return _cbow_sc(input_ids.astype(jnp.int32), table_rm)
